# baseline (speedup 1.0000x reference)
import functools

import jax
import jax.numpy as jnp
from jax import lax
from jax.experimental import pallas as pl
from jax.experimental.pallas import tpu as pltpu
from jax.experimental.pallas import tpu_sc as plsc

BATCH = 16384
EMBED_DIM = 32
HIDDEN_DIM = 64
CHUNK = 128

_info = plsc.get_sparse_core_info()
NC, NS = _info.num_cores, _info.num_subcores
NW = NC * NS
B_PER_W = BATCH // NW
NCH = B_PER_W // CHUNK


def _sc_gather(user_table, movie_table, uidx, m1idx, m2idx):
    mesh = plsc.VectorSubcoreMesh(core_axis_name="c", subcore_axis_name="s")
    out_t = jax.ShapeDtypeStruct((BATCH, EMBED_DIM), jnp.float32)

    @functools.partial(
        pl.kernel,
        mesh=mesh,
        out_type=[out_t, out_t, out_t],
        scratch_types=[
            pltpu.VMEM((B_PER_W,), jnp.int32),
            pltpu.VMEM((B_PER_W,), jnp.int32),
            pltpu.VMEM((B_PER_W,), jnp.int32),
            pltpu.SemaphoreType.DMA,
        ],
    )
    def k(ut_hbm, mt_hbm, ui_hbm, m1i_hbm, m2i_hbm,
          u_out, m1_out, m2_out,
          ui_v, m1i_v, m2i_v, sem):
        wid = lax.axis_index("s") * NC + lax.axis_index("c")
        base = wid * B_PER_W
        pltpu.sync_copy(ui_hbm.at[wid], ui_v)
        pltpu.sync_copy(m1i_hbm.at[wid], m1i_v)
        pltpu.sync_copy(m2i_hbm.at[wid], m2i_v)

        def body(c, _):
            off = c * 16
            uvec = ui_v[pl.ds(off, 16)]
            m1vec = m1i_v[pl.ds(off, 16)]
            m2vec = m2i_v[pl.ds(off, 16)]
            for lane in range(16):
                dst = base + off + lane
                pltpu.async_copy(ut_hbm.at[uvec[lane]], u_out.at[dst], sem)
                pltpu.async_copy(mt_hbm.at[m1vec[lane]], m1_out.at[dst], sem)
                pltpu.async_copy(mt_hbm.at[m2vec[lane]], m2_out.at[dst], sem)
            return 0

        lax.fori_loop(0, B_PER_W // 16, body, 0)
        osl = pl.ds(base, B_PER_W)
        pltpu.make_async_copy(u_out.at[osl], m1_out.at[osl], sem).wait()
        pltpu.make_async_copy(u_out.at[osl], m1_out.at[osl], sem).wait()
        pltpu.make_async_copy(u_out.at[osl], m1_out.at[osl], sem).wait()

    return k(user_table, movie_table, uidx, m1idx, m2idx)


_BLK = 2048


def _mlp_body(u_ref, m1_ref, m2_ref, w1u_ref, w1m_ref, b1_ref, w2_ref, out_ref):
    w1u = w1u_ref[...]
    w1m = w1m_ref[...]
    U = jnp.dot(u_ref[...], w1u, preferred_element_type=jnp.float32)
    M1 = jnp.dot(m1_ref[...], w1m, preferred_element_type=jnp.float32)
    M2 = jnp.dot(m2_ref[...], w1m, preferred_element_type=jnp.float32)
    b1r = b1_ref[...]
    h1 = jnp.maximum(U + M1 + b1r, 0.0)
    h2 = jnp.maximum(U + M2 + b1r, 0.0)
    out_ref[...] = jnp.sum((h1 - h2) * w2_ref[...], axis=1, keepdims=True)


def _tc_mlp(u_emb, m1_emb, m2_emb, W1, b1, W2):
    w1u = W1[:EMBED_DIM]
    w1m = W1[EMBED_DIM:]
    b1r = b1.reshape(1, HIDDEN_DIM)
    w2r = W2.reshape(1, HIDDEN_DIM)
    grid = (BATCH // _BLK,)
    return pl.pallas_call(
        _mlp_body,
        grid=grid,
        in_specs=[
            pl.BlockSpec((_BLK, EMBED_DIM), lambda i: (i, 0)),
            pl.BlockSpec((_BLK, EMBED_DIM), lambda i: (i, 0)),
            pl.BlockSpec((_BLK, EMBED_DIM), lambda i: (i, 0)),
            pl.BlockSpec((EMBED_DIM, HIDDEN_DIM), lambda i: (0, 0)),
            pl.BlockSpec((EMBED_DIM, HIDDEN_DIM), lambda i: (0, 0)),
            pl.BlockSpec((1, HIDDEN_DIM), lambda i: (0, 0)),
            pl.BlockSpec((1, HIDDEN_DIM), lambda i: (0, 0)),
        ],
        out_specs=pl.BlockSpec((_BLK, 1), lambda i: (i, 0)),
        out_shape=jax.ShapeDtypeStruct((BATCH, 1), jnp.float32),
    )(u_emb, m1_emb, m2_emb, w1u, w1m, b1r, w2r)


def kernel(user_ids, movie_ids_1, movie_ids_2, user_table, movie_table,
           W1, b1, W2, b2):
    uidx = user_ids.astype(jnp.int32).reshape(NW, B_PER_W)
    m1idx = movie_ids_1.astype(jnp.int32).reshape(NW, B_PER_W)
    m2idx = movie_ids_2.astype(jnp.int32).reshape(NW, B_PER_W)
    u_emb, m1_emb, m2_emb = _sc_gather(user_table, movie_table,
                                       uidx, m1idx, m2idx)
    return _tc_mlp(u_emb, m1_emb, m2_emb, W1, b1, W2)


# speedup vs baseline: 1.2196x; 1.2196x over previous
import functools

import jax
import jax.numpy as jnp
from jax import lax
from jax.experimental import pallas as pl
from jax.experimental.pallas import tpu as pltpu
from jax.experimental.pallas import tpu_sc as plsc

BATCH = 16384
EMBED_DIM = 32
HIDDEN_DIM = 64
CHUNK = 128
PACK = 128 // EMBED_DIM

_info = plsc.get_sparse_core_info()
NC, NS = _info.num_cores, _info.num_subcores
NW = NC * NS
B_PER_W = BATCH // NW
NCH = B_PER_W // CHUNK


def _sc_gather(ut, mt, uidx, m1idx, m2idx):
    mesh = plsc.VectorSubcoreMesh(core_axis_name="c", subcore_axis_name="s")
    out_t = jax.ShapeDtypeStruct((BATCH, 128), jnp.int32)

    @functools.partial(
        pl.kernel,
        mesh=mesh,
        out_type=[out_t, out_t, out_t],
        scratch_types=[
            pltpu.VMEM((NCH, CHUNK), jnp.int32),
            pltpu.VMEM((NCH, CHUNK), jnp.int32),
            pltpu.VMEM((NCH, CHUNK), jnp.int32),
        ] + [pltpu.VMEM((CHUNK, 128), jnp.int32) for _ in range(6)]
          + [pltpu.SemaphoreType.DMA for _ in range(6)],
    )
    def k(ut_hbm, mt_hbm, ui_hbm, m1i_hbm, m2i_hbm,
          u_out, m1_out, m2_out,
          ui_v, m1i_v, m2i_v, b0, b1_, b2_, b3, b4, b5,
          s0, s1, s2, s3, s4, s5):
        wid = lax.axis_index("s") * NC + lax.axis_index("c")
        base = wid * B_PER_W
        pltpu.sync_copy(ui_hbm.at[wid], ui_v)
        pltpu.sync_copy(m1i_hbm.at[wid], m1i_v)
        pltpu.sync_copy(m2i_hbm.at[wid], m2i_v)
        tabs = [(ut_hbm, ui_v, u_out), (mt_hbm, m1i_v, m1_out),
                (mt_hbm, m2i_v, m2_out)]
        bufs = [b0, b1_, b2_, b3, b4, b5]
        sems = [s0, s1, s2, s3, s4, s5]
        gd = {}
        for t in range(3):
            for s in range(2):
                tbl, idxv, _ = tabs[t]
                gd[(t, s)] = pltpu.async_copy(
                    tbl.at[idxv.at[s]], bufs[2 * t + s], sems[2 * t + s])
        od = {}
        for rnd in range(NCH // 2):
            for t in range(3):
                for s in range(2):
                    ch = 2 * rnd + s
                    tbl, idxv, out = tabs[t]
                    gd[(t, s)].wait()
                    od[(t, s)] = pltpu.async_copy(
                        bufs[2 * t + s],
                        out.at[pl.ds(base + ch * CHUNK, CHUNK)],
                        sems[2 * t + s])
            if rnd + 1 < NCH // 2:
                for t in range(3):
                    for s in range(2):
                        tbl, idxv, _ = tabs[t]
                        od[(t, s)].wait()
                        gd[(t, s)] = pltpu.async_copy(
                            tbl.at[idxv.at[2 * (rnd + 1) + s]],
                            bufs[2 * t + s], sems[2 * t + s])
        for t in range(3):
            for s in range(2):
                od[(t, s)].wait()

    return k(ut, mt, uidx, m1idx, m2idx)


_BLK = 2048


def _mlp_body(u_ref, m1_ref, m2_ref, us_ref, m1s_ref, m2s_ref,
              w1u_ref, w1m_ref, b1_ref, w2_ref, out_ref):
    def pick(x4, sel):
        r = jnp.where(sel == 0, x4[:, 0 * EMBED_DIM:1 * EMBED_DIM], 0.0)
        for kk in range(1, PACK):
            r = r + jnp.where(sel == kk,
                              x4[:, kk * EMBED_DIM:(kk + 1) * EMBED_DIM], 0.0)
        return r

    u = pick(u_ref[...], us_ref[...])
    m1 = pick(m1_ref[...], m1s_ref[...])
    m2 = pick(m2_ref[...], m2s_ref[...])
    U = jnp.dot(u, w1u_ref[...], preferred_element_type=jnp.float32)
    M1 = jnp.dot(m1, w1m_ref[...], preferred_element_type=jnp.float32)
    M2 = jnp.dot(m2, w1m_ref[...], preferred_element_type=jnp.float32)
    b1r = b1_ref[...]
    h1 = jnp.maximum(U + M1 + b1r, 0.0)
    h2 = jnp.maximum(U + M2 + b1r, 0.0)
    out_ref[...] = jnp.sum((h1 - h2) * w2_ref[...], axis=1, keepdims=True)


def _tc_mlp(u4, m14, m24, usel, m1sel, m2sel, W1, b1, W2):
    w1u = W1[:EMBED_DIM]
    w1m = W1[EMBED_DIM:]
    b1r = b1.reshape(1, HIDDEN_DIM)
    w2r = W2.reshape(1, HIDDEN_DIM)
    grid = (BATCH // _BLK,)
    return pl.pallas_call(
        _mlp_body,
        grid=grid,
        in_specs=[
            pl.BlockSpec((_BLK, 128), lambda i: (i, 0)),
            pl.BlockSpec((_BLK, 128), lambda i: (i, 0)),
            pl.BlockSpec((_BLK, 128), lambda i: (i, 0)),
            pl.BlockSpec((_BLK, 1), lambda i: (i, 0)),
            pl.BlockSpec((_BLK, 1), lambda i: (i, 0)),
            pl.BlockSpec((_BLK, 1), lambda i: (i, 0)),
            pl.BlockSpec((EMBED_DIM, HIDDEN_DIM), lambda i: (0, 0)),
            pl.BlockSpec((EMBED_DIM, HIDDEN_DIM), lambda i: (0, 0)),
            pl.BlockSpec((1, HIDDEN_DIM), lambda i: (0, 0)),
            pl.BlockSpec((1, HIDDEN_DIM), lambda i: (0, 0)),
        ],
        out_specs=pl.BlockSpec((_BLK, 1), lambda i: (i, 0)),
        out_shape=jax.ShapeDtypeStruct((BATCH, 1), jnp.float32),
    )(u4, m14, m24, usel, m1sel, m2sel, w1u, w1m, b1r, w2r)


def kernel(user_ids, movie_ids_1, movie_ids_2, user_table, movie_table,
           W1, b1, W2, b2):
    uid = user_ids.astype(jnp.int32)
    m1id = movie_ids_1.astype(jnp.int32)
    m2id = movie_ids_2.astype(jnp.int32)
    uidx = (uid // PACK).reshape(NW, NCH, CHUNK)
    m1idx = (m1id // PACK).reshape(NW, NCH, CHUNK)
    m2idx = (m2id // PACK).reshape(NW, NCH, CHUNK)
    ut = lax.bitcast_convert_type(user_table, jnp.int32).reshape(-1, 128)
    mt = lax.bitcast_convert_type(movie_table, jnp.int32).reshape(-1, 128)
    u4, m14, m24 = _sc_gather(ut, mt, uidx, m1idx, m2idx)

    def _f32(x):
        return lax.bitcast_convert_type(x, jnp.float32)

    return _tc_mlp(_f32(u4), _f32(m14), _f32(m24),
                   (uid % PACK).reshape(BATCH, 1),
                   (m1id % PACK).reshape(BATCH, 1),
                   (m2id % PACK).reshape(BATCH, 1),
                   W1, b1, W2)
